# baseline (speedup 1.0000x reference)
import functools

import jax
import jax.numpy as jnp
from jax import lax
from jax.sharding import PartitionSpec as P
from jax.experimental import pallas as pl
from jax.experimental.pallas import tpu as pltpu
from jax.experimental.pallas import tpu_sc as plsc

NC_PTS = 4096
NF_PTS = 16384
CIN = 512
COUT = 256
KNN = 3


def _h2_body(feats_ref, g_ref, b_ref, w2_ref, b2_ref, out_ref):
    x = feats_ref[...]
    m = jnp.mean(x, axis=-1, keepdims=True)
    v = jnp.mean((x - m) * (x - m), axis=-1, keepdims=True)
    xn = (x - m) / jnp.sqrt(v + 1e-5) * g_ref[...] + b_ref[...]
    h2 = (jnp.dot(xn, w2_ref[...], preferred_element_type=jnp.float32)
          + b2_ref[...])
    a = h2[:, :COUT // 2]
    b = h2[:, COUT // 2:]
    pa = lax.bitcast_convert_type(
        a.astype(jnp.bfloat16).astype(jnp.float32), jnp.uint32)
    pb = lax.bitcast_convert_type(
        b.astype(jnp.bfloat16).astype(jnp.float32), jnp.uint32)
    packed = jnp.bitwise_or(jnp.right_shift(pa, jnp.uint32(16)),
                            jnp.bitwise_and(pb, jnp.uint32(0xFFFF0000)))
    out_ref[...] = lax.bitcast_convert_type(packed, jnp.float32)


def _knn_body(sxyz_ref, xyzt_ref, idx_ref, w_ref):
    s = sxyz_ref[...]
    x = xyzt_ref[...]
    br = s.shape[0]
    s2 = jnp.sum(s * s, axis=1, keepdims=True)
    x2 = jnp.sum(x * x, axis=0, keepdims=True)
    sb = s.astype(jnp.bfloat16).astype(jnp.float32)
    xb = x.astype(jnp.bfloat16).astype(jnp.float32)
    dot = jnp.dot(sb, xb, preferred_element_type=jnp.float32)
    d = s2 + x2 - 2.0 * dot

    colsf = lax.broadcasted_iota(jnp.int32, (br, NC_PTS), 1).astype(jnp.float32)
    vals = []
    idxs = []
    for k in range(KNN):
        m = jnp.min(d, axis=1, keepdims=True)
        i = jnp.min(jnp.where(d == m, colsf, jnp.float32(NC_PTS)),
                    axis=1, keepdims=True)
        vals.append(m)
        idxs.append(i.astype(jnp.int32))
        if k < KNN - 1:
            d = jnp.where(colsf == i, jnp.inf, d)

    dist = [jnp.sqrt(jnp.maximum(v, 0.0)) for v in vals]
    u = [1.0 / (dk + 1e-8) for dk in dist]
    usum = u[0] + u[1] + u[2]
    w = [uk / usum for uk in u]

    idx_ref[...] = jnp.concatenate(idxs, axis=1)
    w_ref[...] = jnp.concatenate(w, axis=1)


_GATHER_WINDOW = 384


def _sc_gather(h2_packed, idx_flat):
    num_idx = idx_flat.shape[1]
    ncols = h2_packed.shape[1]
    mesh = plsc.VectorSubcoreMesh(core_axis_name="core",
                                  subcore_axis_name="subcore")

    @functools.partial(
        pl.kernel,
        out_type=jax.ShapeDtypeStruct((num_idx, ncols), jnp.float32),
        mesh=mesh,
    )
    def gather_kernel(h2_hbm, i_hbm, o_hbm):
        def body(i_vmem, o_vmem):
            pltpu.sync_copy(h2_hbm.at[i_vmem.at[0]], o_vmem)

        pltpu.emit_pipeline(
            body,
            grid=(num_idx // _GATHER_WINDOW,),
            in_specs=[pl.BlockSpec((1, _GATHER_WINDOW),
                                   index_map=lambda i: (0, i))],
            out_specs=[pl.BlockSpec((_GATHER_WINDOW, ncols),
                                    index_map=lambda i: (i, 0))],
            core_axis_name=("core", "subcore"),
            dimension_semantics=(pltpu.PARALLEL,),
        )(i_hbm, o_hbm)

    return gather_kernel(h2_packed, idx_flat)


def _final_body(sf_ref, g_ref, b_ref, w1_ref, b1_ref, w_ref,
                g0_ref, g1_ref, g2_ref, out_ref):
    x = sf_ref[...]
    m = jnp.mean(x, axis=-1, keepdims=True)
    v = jnp.mean((x - m) * (x - m), axis=-1, keepdims=True)
    xn = (x - m) / jnp.sqrt(v + 1e-5) * g_ref[...] + b_ref[...]
    h1 = (jnp.dot(xn, w1_ref[...], preferred_element_type=jnp.float32)
          + b1_ref[...])

    def unpack(g_ref):
        u = lax.bitcast_convert_type(g_ref[...], jnp.uint32)
        lo = lax.bitcast_convert_type(
            jnp.left_shift(u, jnp.uint32(16)), jnp.float32)
        hi = lax.bitcast_convert_type(
            jnp.bitwise_and(u, jnp.uint32(0xFFFF0000)), jnp.float32)
        return lo, hi

    lo0, hi0 = unpack(g0_ref)
    lo1, hi1 = unpack(g1_ref)
    lo2, hi2 = unpack(g2_ref)
    w0 = w_ref[:, 0:1]
    w1 = w_ref[:, 1:2]
    w2 = w_ref[:, 2:3]
    interp_lo = w0 * lo0 + w1 * lo1 + w2 * lo2
    interp_hi = w0 * hi0 + w1 * hi1 + w2 * hi2
    out_ref[...] = h1 + jnp.concatenate([interp_lo, interp_hi], axis=1)


def _knn_call(sxyz, xyzt):
    nf = sxyz.shape[0]
    BR = 1024
    return pl.pallas_call(
        _knn_body,
        grid=(nf // BR,),
        in_specs=[
            pl.BlockSpec((BR, 3), lambda i: (i, 0)),
            pl.BlockSpec((3, NC_PTS), lambda i: (0, 0)),
        ],
        out_specs=[
            pl.BlockSpec((BR, KNN), lambda i: (i, 0)),
            pl.BlockSpec((BR, KNN), lambda i: (i, 0)),
        ],
        out_shape=[
            jax.ShapeDtypeStruct((nf, KNN), jnp.int32),
            jax.ShapeDtypeStruct((nf, KNN), jnp.float32),
        ],
    )(sxyz, xyzt)


def _final_call(sfeats, ln1_g, ln1_b, W1, b1, w3, gathered):
    nf = sfeats.shape[0]
    BF = 2048
    nsteps = nf // BF
    return pl.pallas_call(
        _final_body,
        grid=(nsteps,),
        in_specs=[
            pl.BlockSpec((BF, COUT), lambda i: (i, 0)),
            pl.BlockSpec((COUT,), lambda i: (0,)),
            pl.BlockSpec((COUT,), lambda i: (0,)),
            pl.BlockSpec((COUT, COUT), lambda i: (0, 0)),
            pl.BlockSpec((COUT,), lambda i: (0,)),
            pl.BlockSpec((BF, KNN), lambda i: (i, 0)),
            pl.BlockSpec((BF, COUT // 2), lambda i: (i, 0)),
            pl.BlockSpec((BF, COUT // 2), lambda i: (nsteps + i, 0)),
            pl.BlockSpec((BF, COUT // 2), lambda i: (2 * nsteps + i, 0)),
        ],
        out_specs=pl.BlockSpec((BF, COUT), lambda i: (i, 0)),
        out_shape=jax.ShapeDtypeStruct((nf, COUT), jnp.float32),
    )(sfeats, ln1_g, ln1_b, W1, b1, w3,
      gathered, gathered, gathered)


def _local_pipeline(feats, xyzt, sxyz, sfeats,
                    ln1_g, ln1_b, W1, b1, ln2_g, ln2_b, W2, b2):
    nf = sxyz.shape[0]

    h2p = pl.pallas_call(
        _h2_body,
        out_shape=jax.ShapeDtypeStruct((NC_PTS, COUT // 2), jnp.float32),
    )(feats, ln2_g, ln2_b, W2, b2)

    NHALF = 1
    nh = nf // NHALF
    outs = []
    stage = []
    for p in range(NHALF):
        sl = slice(p * nh, (p + 1) * nh)
        idx3, w3 = _knn_call(sxyz[sl], xyzt)
        gathered = _sc_gather(h2p, idx3.T.reshape(1, KNN * nh))
        stage.append((sl, w3, gathered))
    for sl, w3, gathered in stage:
        outs.append(_final_call(sfeats[sl], ln1_g, ln1_b, W1, b1,
                                w3, gathered))
    if NHALF == 1:
        return outs[0]
    return jnp.concatenate(outs, axis=0)


def kernel(feats, xyz, support_xyz, offset, support_offset, support_feats,
           ln1_g, ln1_b, W1, b1, ln2_g, ln2_b, W2, b2):
    xyzt = xyz.T

    ndev = min(2, jax.device_count())
    if ndev > 1 and NF_PTS % ndev == 0:
        mesh = jax.make_mesh((ndev,), ("d",))
        rep = P()
        sxyz_sh = jax.reshard(support_xyz,
                              jax.NamedSharding(mesh, P("d")))
        sfeats_sh = jax.reshard(support_feats,
                                jax.NamedSharding(mesh, P("d")))
        out = jax.shard_map(
            _local_pipeline,
            mesh=mesh,
            in_specs=(rep, rep, P("d"), P("d"),
                      rep, rep, rep, rep, rep, rep, rep, rep),
            out_specs=P("d"),
            check_vma=False,
        )(feats, xyzt, sxyz_sh, sfeats_sh,
          ln1_g, ln1_b, W1, b1, ln2_g, ln2_b, W2, b2)
    else:
        out = _local_pipeline(feats, xyzt, support_xyz, support_feats,
                              ln1_g, ln1_b, W1, b1, ln2_g, ln2_b, W2, b2)

    return (out, support_xyz, support_offset)


# speedup vs baseline: 1.5445x; 1.5445x over previous
import functools

import jax
import jax.numpy as jnp
from jax import lax
from jax.experimental import pallas as pl
from jax.experimental.pallas import tpu as pltpu
from jax.experimental.pallas import tpu_sc as plsc

NC_PTS = 4096
NF_PTS = 16384
CIN = 512
COUT = 256
KNN = 3


def _h2_body(feats_ref, g_ref, b_ref, w2_ref, b2_ref, out_ref):
    x = feats_ref[...]
    m = jnp.mean(x, axis=-1, keepdims=True)
    v = jnp.mean((x - m) * (x - m), axis=-1, keepdims=True)
    xn = (x - m) / jnp.sqrt(v + 1e-5) * g_ref[...] + b_ref[...]
    h2 = (jnp.dot(xn, w2_ref[...], preferred_element_type=jnp.float32)
          + b2_ref[...])
    a = h2[:, :COUT // 2]
    b = h2[:, COUT // 2:]
    pa = lax.bitcast_convert_type(
        a.astype(jnp.bfloat16).astype(jnp.float32), jnp.uint32)
    pb = lax.bitcast_convert_type(
        b.astype(jnp.bfloat16).astype(jnp.float32), jnp.uint32)
    packed = jnp.bitwise_or(jnp.right_shift(pa, jnp.uint32(16)),
                            jnp.bitwise_and(pb, jnp.uint32(0xFFFF0000)))
    out_ref[...] = lax.bitcast_convert_type(packed, jnp.float32)


def _knn_body(sxyz_ref, xyzt_ref, cols_ref, idx_ref, w_ref):
    s = sxyz_ref[...]
    x = xyzt_ref[...]
    s2 = jnp.sum(s * s, axis=1, keepdims=True)
    x2 = jnp.sum(x * x, axis=0, keepdims=True)
    sb = s.astype(jnp.bfloat16).astype(jnp.float32)
    xb = x.astype(jnp.bfloat16).astype(jnp.float32)
    dot = jnp.dot(sb, xb, preferred_element_type=jnp.float32)
    d = s2 + x2 - 2.0 * dot

    colsf = cols_ref[...]
    vals = []
    idxs = []
    for k in range(KNN):
        m = jnp.min(d, axis=1, keepdims=True)
        i = jnp.min(jnp.where(d == m, colsf, jnp.float32(NC_PTS)),
                    axis=1, keepdims=True)
        vals.append(m)
        idxs.append(i.astype(jnp.int32))
        if k < KNN - 1:
            d = jnp.where(colsf == i, jnp.inf, d)

    dist = [jnp.sqrt(jnp.maximum(v, 0.0)) for v in vals]
    u = [1.0 / (dk + 1e-8) for dk in dist]
    usum = u[0] + u[1] + u[2]
    w = [uk / usum for uk in u]

    idx_ref[...] = jnp.concatenate(idxs, axis=1)
    w_ref[...] = jnp.concatenate(w, axis=1)


_GATHER_WINDOW = 384


def _sc_gather(h2_packed, idx_flat):
    num_idx = idx_flat.shape[1]
    ncols = h2_packed.shape[1]
    mesh = plsc.VectorSubcoreMesh(core_axis_name="core",
                                  subcore_axis_name="subcore")

    @functools.partial(
        pl.kernel,
        out_type=jax.ShapeDtypeStruct((num_idx, ncols), jnp.float32),
        mesh=mesh,
    )
    def gather_kernel(h2_hbm, i_hbm, o_hbm):
        def body(i_vmem, o_vmem):
            pltpu.sync_copy(h2_hbm.at[i_vmem.at[0]], o_vmem)

        pltpu.emit_pipeline(
            body,
            grid=(num_idx // _GATHER_WINDOW,),
            in_specs=[pl.BlockSpec((1, _GATHER_WINDOW),
                                   index_map=lambda i: (0, i))],
            out_specs=[pl.BlockSpec((_GATHER_WINDOW, ncols),
                                    index_map=lambda i: (i, 0))],
            core_axis_name=("core", "subcore"),
            dimension_semantics=(pltpu.PARALLEL,),
        )(i_hbm, o_hbm)

    return gather_kernel(h2_packed, idx_flat)


def _final_body(sf_ref, g_ref, b_ref, w1_ref, b1_ref, w_ref,
                g0_ref, g1_ref, g2_ref, out_ref):
    x = sf_ref[...]
    m = jnp.mean(x, axis=-1, keepdims=True)
    v = jnp.mean((x - m) * (x - m), axis=-1, keepdims=True)
    xn = (x - m) / jnp.sqrt(v + 1e-5) * g_ref[...] + b_ref[...]
    h1 = (jnp.dot(xn, w1_ref[...], preferred_element_type=jnp.float32)
          + b1_ref[...])

    def unpack(g_ref):
        u = lax.bitcast_convert_type(g_ref[...], jnp.uint32)
        lo = lax.bitcast_convert_type(
            jnp.left_shift(u, jnp.uint32(16)), jnp.float32)
        hi = lax.bitcast_convert_type(
            jnp.bitwise_and(u, jnp.uint32(0xFFFF0000)), jnp.float32)
        return lo, hi

    lo0, hi0 = unpack(g0_ref)
    lo1, hi1 = unpack(g1_ref)
    lo2, hi2 = unpack(g2_ref)
    w0 = w_ref[:, 0:1]
    w1 = w_ref[:, 1:2]
    w2 = w_ref[:, 2:3]
    interp_lo = w0 * lo0 + w1 * lo1 + w2 * lo2
    interp_hi = w0 * hi0 + w1 * hi1 + w2 * hi2
    out_ref[...] = h1 + jnp.concatenate([interp_lo, interp_hi], axis=1)


def _knn_call(sxyz, xyzt, cols_row):
    nf = sxyz.shape[0]
    BR = 1024
    return pl.pallas_call(
        _knn_body,
        grid=(nf // BR,),
        in_specs=[
            pl.BlockSpec((BR, 3), lambda i: (i, 0)),
            pl.BlockSpec((3, NC_PTS), lambda i: (0, 0)),
            pl.BlockSpec((1, NC_PTS), lambda i: (0, 0)),
        ],
        out_specs=[
            pl.BlockSpec((BR, KNN), lambda i: (i, 0)),
            pl.BlockSpec((BR, KNN), lambda i: (i, 0)),
        ],
        out_shape=[
            jax.ShapeDtypeStruct((nf, KNN), jnp.int32),
            jax.ShapeDtypeStruct((nf, KNN), jnp.float32),
        ],
    )(sxyz, xyzt, cols_row)


def _final_call(sfeats, ln1_g, ln1_b, W1, b1, w3, gathered):
    nf = sfeats.shape[0]
    BF = 2048
    nsteps = nf // BF
    return pl.pallas_call(
        _final_body,
        grid=(nsteps,),
        in_specs=[
            pl.BlockSpec((BF, COUT), lambda i: (i, 0)),
            pl.BlockSpec((COUT,), lambda i: (0,)),
            pl.BlockSpec((COUT,), lambda i: (0,)),
            pl.BlockSpec((COUT, COUT), lambda i: (0, 0)),
            pl.BlockSpec((COUT,), lambda i: (0,)),
            pl.BlockSpec((BF, KNN), lambda i: (i, 0)),
            pl.BlockSpec((BF, COUT // 2), lambda i: (i, 0)),
            pl.BlockSpec((BF, COUT // 2), lambda i: (nsteps + i, 0)),
            pl.BlockSpec((BF, COUT // 2), lambda i: (2 * nsteps + i, 0)),
        ],
        out_specs=pl.BlockSpec((BF, COUT), lambda i: (i, 0)),
        out_shape=jax.ShapeDtypeStruct((nf, COUT), jnp.float32),
    )(sfeats, ln1_g, ln1_b, W1, b1, w3,
      gathered, gathered, gathered)


def kernel(feats, xyz, support_xyz, offset, support_offset, support_feats,
           ln1_g, ln1_b, W1, b1, ln2_g, ln2_b, W2, b2):
    xyzt = xyz.T
    cols_row = jnp.arange(NC_PTS, dtype=jnp.float32).reshape(1, NC_PTS)

    h2p = pl.pallas_call(
        _h2_body,
        out_shape=jax.ShapeDtypeStruct((NC_PTS, COUT // 2), jnp.float32),
    )(feats, ln2_g, ln2_b, W2, b2)

    idx3, w3 = _knn_call(support_xyz, xyzt, cols_row)
    gathered = _sc_gather(h2p, idx3.T.reshape(1, KNN * NF_PTS))
    out = _final_call(support_feats, ln1_g, ln1_b, W1, b1, w3, gathered)

    return (out, support_xyz, support_offset)


# speedup vs baseline: 1.5648x; 1.0131x over previous
import functools

import jax
import jax.numpy as jnp
from jax import lax
from jax.experimental import pallas as pl
from jax.experimental.pallas import tpu as pltpu
from jax.experimental.pallas import tpu_sc as plsc

NC_PTS = 4096
NF_PTS = 16384
CIN = 512
COUT = 256
KNN = 3


def _h2_body(feats_ref, g_ref, b_ref, w2_ref, b2_ref, out_ref):
    x = feats_ref[...]
    m = jnp.mean(x, axis=-1, keepdims=True)
    v = jnp.mean((x - m) * (x - m), axis=-1, keepdims=True)
    xn = (x - m) / jnp.sqrt(v + 1e-5) * g_ref[...] + b_ref[...]
    h2 = (jnp.dot(xn, w2_ref[...], preferred_element_type=jnp.float32)
          + b2_ref[...])
    a = h2[:, :COUT // 2]
    b = h2[:, COUT // 2:]
    pa = lax.bitcast_convert_type(
        a.astype(jnp.bfloat16).astype(jnp.float32), jnp.uint32)
    pb = lax.bitcast_convert_type(
        b.astype(jnp.bfloat16).astype(jnp.float32), jnp.uint32)
    packed = jnp.bitwise_or(jnp.right_shift(pa, jnp.uint32(16)),
                            jnp.bitwise_and(pb, jnp.uint32(0xFFFF0000)))
    out_ref[...] = lax.bitcast_convert_type(packed, jnp.float32)


def _knn_body(sxyz_ref, xyzt_ref, cols_ref, idx_ref, w_ref):
    s = sxyz_ref[...]
    x = xyzt_ref[...]
    s2 = jnp.sum(s * s, axis=1, keepdims=True)
    x2 = jnp.sum(x * x, axis=0, keepdims=True)
    sb = s.astype(jnp.bfloat16).astype(jnp.float32)
    xb = x.astype(jnp.bfloat16).astype(jnp.float32)
    dot = jnp.dot(sb, xb, preferred_element_type=jnp.float32)
    d = s2 + x2 - 2.0 * dot

    colsf = cols_ref[...]
    vals = []
    idxs = []
    for k in range(KNN):
        m = jnp.min(d, axis=1, keepdims=True)
        i = jnp.min(jnp.where(d == m, colsf, jnp.float32(NC_PTS)),
                    axis=1, keepdims=True)
        vals.append(m)
        idxs.append(i.astype(jnp.int32))
        if k < KNN - 1:
            d = jnp.where(colsf == i, jnp.inf, d)

    dist = [jnp.sqrt(jnp.maximum(v, 0.0)) for v in vals]
    u = [1.0 / (dk + 1e-8) for dk in dist]
    usum = u[0] + u[1] + u[2]
    w = [uk / usum for uk in u]

    idx_ref[...] = jnp.concatenate(idxs, axis=1)
    w_ref[...] = jnp.concatenate(w, axis=1)


_GATHER_WINDOW = 384


def _sc_gather(h2_packed, idx_flat):
    num_idx = idx_flat.shape[1]
    ncols = h2_packed.shape[1]
    mesh = plsc.VectorSubcoreMesh(core_axis_name="core",
                                  subcore_axis_name="subcore")

    @functools.partial(
        pl.kernel,
        out_type=jax.ShapeDtypeStruct((num_idx, ncols), jnp.float32),
        mesh=mesh,
    )
    def gather_kernel(h2_hbm, i_hbm, o_hbm):
        def body(i_vmem, o_vmem):
            pltpu.sync_copy(h2_hbm.at[i_vmem.at[0]], o_vmem)

        pltpu.emit_pipeline(
            body,
            grid=(num_idx // _GATHER_WINDOW,),
            in_specs=[pl.BlockSpec((1, _GATHER_WINDOW),
                                   index_map=lambda i: (0, i))],
            out_specs=[pl.BlockSpec((_GATHER_WINDOW, ncols),
                                    index_map=lambda i: (i, 0))],
            core_axis_name=("core", "subcore"),
            dimension_semantics=(pltpu.PARALLEL,),
        )(i_hbm, o_hbm)

    return gather_kernel(h2_packed, idx_flat)


def _final_body(sf_ref, g_ref, b_ref, w1_ref, b1_ref, w_ref,
                g0_ref, g1_ref, g2_ref, out_ref):
    x = sf_ref[...]
    m = jnp.mean(x, axis=-1, keepdims=True)
    v = jnp.mean((x - m) * (x - m), axis=-1, keepdims=True)
    xn = (x - m) / jnp.sqrt(v + 1e-5) * g_ref[...] + b_ref[...]
    h1 = (jnp.dot(xn, w1_ref[...], preferred_element_type=jnp.float32)
          + b1_ref[...])

    def unpack(g_ref):
        u = lax.bitcast_convert_type(g_ref[...], jnp.uint32)
        lo = lax.bitcast_convert_type(
            jnp.left_shift(u, jnp.uint32(16)), jnp.float32)
        hi = lax.bitcast_convert_type(
            jnp.bitwise_and(u, jnp.uint32(0xFFFF0000)), jnp.float32)
        return lo, hi

    lo0, hi0 = unpack(g0_ref)
    lo1, hi1 = unpack(g1_ref)
    lo2, hi2 = unpack(g2_ref)
    w0 = w_ref[:, 0:1]
    w1 = w_ref[:, 1:2]
    w2 = w_ref[:, 2:3]
    interp_lo = w0 * lo0 + w1 * lo1 + w2 * lo2
    interp_hi = w0 * hi0 + w1 * hi1 + w2 * hi2
    out_ref[...] = h1 + jnp.concatenate([interp_lo, interp_hi], axis=1)


def _knn_call(sxyz, xyzt, cols_row):
    nf = sxyz.shape[0]
    BR = 2048
    return pl.pallas_call(
        _knn_body,
        grid=(nf // BR,),
        in_specs=[
            pl.BlockSpec((BR, 3), lambda i: (i, 0)),
            pl.BlockSpec((3, NC_PTS), lambda i: (0, 0)),
            pl.BlockSpec((1, NC_PTS), lambda i: (0, 0)),
        ],
        out_specs=[
            pl.BlockSpec((BR, KNN), lambda i: (i, 0)),
            pl.BlockSpec((BR, KNN), lambda i: (i, 0)),
        ],
        out_shape=[
            jax.ShapeDtypeStruct((nf, KNN), jnp.int32),
            jax.ShapeDtypeStruct((nf, KNN), jnp.float32),
        ],
    )(sxyz, xyzt, cols_row)


def _final_call(sfeats, ln1_g, ln1_b, W1, b1, w3, gathered):
    nf = sfeats.shape[0]
    BF = 2048
    nsteps = nf // BF
    return pl.pallas_call(
        _final_body,
        grid=(nsteps,),
        in_specs=[
            pl.BlockSpec((BF, COUT), lambda i: (i, 0)),
            pl.BlockSpec((COUT,), lambda i: (0,)),
            pl.BlockSpec((COUT,), lambda i: (0,)),
            pl.BlockSpec((COUT, COUT), lambda i: (0, 0)),
            pl.BlockSpec((COUT,), lambda i: (0,)),
            pl.BlockSpec((BF, KNN), lambda i: (i, 0)),
            pl.BlockSpec((BF, COUT // 2), lambda i: (i, 0)),
            pl.BlockSpec((BF, COUT // 2), lambda i: (nsteps + i, 0)),
            pl.BlockSpec((BF, COUT // 2), lambda i: (2 * nsteps + i, 0)),
        ],
        out_specs=pl.BlockSpec((BF, COUT), lambda i: (i, 0)),
        out_shape=jax.ShapeDtypeStruct((nf, COUT), jnp.float32),
    )(sfeats, ln1_g, ln1_b, W1, b1, w3,
      gathered, gathered, gathered)


def kernel(feats, xyz, support_xyz, offset, support_offset, support_feats,
           ln1_g, ln1_b, W1, b1, ln2_g, ln2_b, W2, b2):
    xyzt = xyz.T
    cols_row = jnp.arange(NC_PTS, dtype=jnp.float32).reshape(1, NC_PTS)

    h2p = pl.pallas_call(
        _h2_body,
        out_shape=jax.ShapeDtypeStruct((NC_PTS, COUT // 2), jnp.float32),
    )(feats, ln2_g, ln2_b, W2, b2)

    idx3, w3 = _knn_call(support_xyz, xyzt, cols_row)
    gathered = _sc_gather(h2p, idx3.T.reshape(1, KNN * NF_PTS))
    out = _final_call(support_feats, ln1_g, ln1_b, W1, b1, w3, gathered)

    return (out, support_xyz, support_offset)
